# baseline (speedup 1.0000x reference)
import functools

import jax
import jax.numpy as jnp
from jax import lax
from jax.experimental import pallas as pl
from jax.experimental.pallas import tpu as pltpu
from jax.experimental.pallas import tpu_sc as plsc

BATCH, SEQ, D = 4, 4096, 2048
NC, NS = 2, 16
NW = NC * NS
SEQ_PER_W = SEQ // NW
CS = 4
NCHUNK = SEQ_PER_W // CS
LANES = 16
VPR = D // LANES


def _sc_body(x_hbm, emb_hbm, out_hbm, xbuf, ebuf):
    wid = lax.axis_index("s") * NC + lax.axis_index("c")
    s_base = wid * SEQ_PER_W

    def chunk(ci, carry):
        s0 = s_base + ci * CS
        pltpu.sync_copy(emb_hbm.at[pl.ds(s0, CS)], ebuf)
        for b in range(BATCH):
            pltpu.sync_copy(x_hbm.at[b, pl.ds(s0, CS)], xbuf.at[b])

        def vec(j, c):
            off = j * LANES
            for s in range(CS):
                e = ebuf[s, pl.ds(off, LANES)]
                for b in range(BATCH):
                    xbuf[b, s, pl.ds(off, LANES)] = (
                        xbuf[b, s, pl.ds(off, LANES)] + e
                    )
            return c

        lax.fori_loop(0, VPR, vec, 0)
        for b in range(BATCH):
            pltpu.sync_copy(xbuf.at[b], out_hbm.at[b, pl.ds(s0, CS)])
        return carry

    lax.fori_loop(0, NCHUNK, chunk, 0)


INTERPRET = False


def kernel(x, embedding):
    mesh = plsc.VectorSubcoreMesh(
        core_axis_name="c", subcore_axis_name="s", num_cores=NC, num_subcores=NS
    )
    f = pl.kernel(
        _sc_body,
        jax.ShapeDtypeStruct((BATCH, SEQ, D), jnp.float32),
        mesh=mesh,
        scratch_types=[
            pltpu.VMEM((BATCH, CS, D), jnp.float32),
            pltpu.VMEM((CS, D), jnp.float32),
        ],
        interpret=INTERPRET,
    )
    return f(x, embedding)


# speedup vs baseline: 2.0766x; 2.0766x over previous
import jax
import jax.numpy as jnp
from jax import lax
from jax.experimental import pallas as pl
from jax.experimental.pallas import tpu as pltpu
from jax.experimental.pallas import tpu_sc as plsc

BATCH, SEQ, D = 4, 4096, 2048
NC, NS = 2, 16
NW = NC * NS
SEQ_PER_W = SEQ // NW
CS = 4
NCHUNK = SEQ_PER_W // CS
NBUF = 2
LANES = 16
VPR = D // LANES


def _sc_body(x_hbm, emb_hbm, out_hbm, xbuf, ebuf, insem, outsem):
    wid = lax.axis_index("s") * NC + lax.axis_index("c")
    s_base = wid * SEQ_PER_W

    in_descs = {}
    out_descs = {}

    def start_in(ci, k):
        s0 = s_base + ci * CS
        d1 = pltpu.async_copy(x_hbm.at[:, pl.ds(s0, CS)], xbuf.at[k], insem.at[k])
        d2 = pltpu.async_copy(emb_hbm.at[pl.ds(s0, CS)], ebuf.at[k], insem.at[k])
        in_descs[k] = (d1, d2)

    def start_out(ci, k):
        s0 = s_base + ci * CS
        out_descs[k] = pltpu.async_copy(
            xbuf.at[k], out_hbm.at[:, pl.ds(s0, CS)], outsem.at[k]
        )

    def compute(k):
        def vec(j, c):
            off = j * LANES
            for s in range(CS):
                e = ebuf[k, s, pl.ds(off, LANES)]
                for b in range(BATCH):
                    xbuf[k, b, s, pl.ds(off, LANES)] = (
                        xbuf[k, b, s, pl.ds(off, LANES)] + e
                    )
            return c

        lax.fori_loop(0, VPR, vec, 0)

    start_in(0, 0)
    for ci in range(NCHUNK):
        k = ci % NBUF
        kn = (ci + 1) % NBUF
        if ci + 1 < NCHUNK:
            if ci >= NBUF - 1:
                out_descs[kn].wait()
            start_in(ci + 1, kn)
        d1, d2 = in_descs[k]
        d1.wait()
        d2.wait()
        compute(k)
        start_out(ci, k)
    for k in range(NBUF):
        if k in out_descs:
            out_descs[k].wait()


INTERPRET = False


def kernel(x, embedding):
    mesh = plsc.VectorSubcoreMesh(
        core_axis_name="c", subcore_axis_name="s", num_cores=NC, num_subcores=NS
    )
    f = pl.kernel(
        _sc_body,
        jax.ShapeDtypeStruct((BATCH, SEQ, D), jnp.float32),
        mesh=mesh,
        scratch_types=[
            pltpu.VMEM((NBUF, BATCH, CS, D), jnp.float32),
            pltpu.VMEM((NBUF, CS, D), jnp.float32),
            pltpu.SemaphoreType.DMA((NBUF,)),
            pltpu.SemaphoreType.DMA((NBUF,)),
        ],
        interpret=INTERPRET,
    )
    return f(x, embedding)


# speedup vs baseline: 2.1809x; 1.0503x over previous
import jax
import jax.numpy as jnp
from jax import lax
from jax.experimental import pallas as pl
from jax.experimental.pallas import tpu as pltpu
from jax.experimental.pallas import tpu_sc as plsc

BATCH, SEQ, D = 4, 4096, 2048
NC, NS = 2, 16
NW = NC * NS
SEQ_PER_W = SEQ // NW
CS = 4
NCHUNK = SEQ_PER_W // CS
NBUF = 3
LANES = 16
VPR = D // LANES


def _sc_body(x_hbm, emb_hbm, out_hbm, xbuf, ebuf, insem, outsem):
    wid = lax.axis_index("s") * NC + lax.axis_index("c")
    s_base = wid * SEQ_PER_W

    def in_copies(ci, k):
        s0 = s_base + ci * CS
        return (
            pltpu.make_async_copy(
                x_hbm.at[:, pl.ds(s0, CS)], xbuf.at[k], insem.at[k]
            ),
            pltpu.make_async_copy(
                emb_hbm.at[pl.ds(s0, CS)], ebuf.at[k], insem.at[k]
            ),
        )

    def out_copy(ci, k):
        s0 = s_base + ci * CS
        return pltpu.make_async_copy(
            xbuf.at[k], out_hbm.at[:, pl.ds(s0, CS)], outsem.at[k]
        )

    def start_in(ci, k):
        for c in in_copies(ci, k):
            c.start()

    def wait_in(ci, k):
        for c in in_copies(ci, k):
            c.wait()

    def compute(k):
        @plsc.parallel_loop(0, VPR, step=1, unroll=2)
        def vec(j):
            off = j * LANES
            for s in range(CS):
                e = ebuf[k, s, pl.ds(off, LANES)]
                for b in range(BATCH):
                    xbuf[k, b, s, pl.ds(off, LANES)] = (
                        xbuf[k, b, s, pl.ds(off, LANES)] + e
                    )

    start_in(0, 0)

    def step(ci, carry):
        k = lax.rem(ci, NBUF)
        kn = lax.rem(ci + 1, NBUF)

        @pl.when(jnp.logical_and(ci + 1 < NCHUNK, ci >= NBUF - 1))
        def _():
            out_copy(ci + 1 - NBUF, kn).wait()

        @pl.when(ci + 1 < NCHUNK)
        def _():
            start_in(ci + 1, kn)

        wait_in(ci, k)
        compute(k)
        out_copy(ci, k).start()
        return carry

    lax.fori_loop(0, NCHUNK, step, 0)
    for ci in range(NCHUNK - NBUF, NCHUNK):
        out_copy(ci, ci % NBUF).wait()


INTERPRET = False


def kernel(x, embedding):
    mesh = plsc.VectorSubcoreMesh(
        core_axis_name="c", subcore_axis_name="s", num_cores=NC, num_subcores=NS
    )
    f = pl.kernel(
        _sc_body,
        jax.ShapeDtypeStruct((BATCH, SEQ, D), jnp.float32),
        mesh=mesh,
        scratch_types=[
            pltpu.VMEM((NBUF, BATCH, CS, D), jnp.float32),
            pltpu.VMEM((NBUF, CS, D), jnp.float32),
            pltpu.SemaphoreType.DMA((NBUF,)),
            pltpu.SemaphoreType.DMA((NBUF,)),
        ],
        interpret=INTERPRET,
    )
    return f(x, embedding)


# speedup vs baseline: 2.1812x; 1.0001x over previous
import jax
import jax.numpy as jnp
from jax import lax
from jax.experimental import pallas as pl
from jax.experimental.pallas import tpu as pltpu
from jax.experimental.pallas import tpu_sc as plsc

BATCH, SEQ, D = 4, 4096, 2048
NC, NS = 2, 16
NW = NC * NS
SEQ_PER_W = SEQ // NW
CS = 4
NCHUNK = SEQ_PER_W // CS
NBUF = 3
LANES = 16
VPR = D // LANES


def _sc_body(x_hbm, emb_hbm, out_hbm, xbuf, ebuf, insem, outsem):
    wid = lax.axis_index("s") * NC + lax.axis_index("c")
    s_base = wid * SEQ_PER_W

    def in_copies(ci, k):
        s0 = s_base + ci * CS
        return (
            pltpu.make_async_copy(
                x_hbm.at[:, pl.ds(s0, CS)], xbuf.at[k], insem.at[k]
            ),
            pltpu.make_async_copy(
                emb_hbm.at[pl.ds(s0, CS)], ebuf.at[k], insem.at[k]
            ),
        )

    def out_copy(ci, k):
        s0 = s_base + ci * CS
        return pltpu.make_async_copy(
            xbuf.at[k], out_hbm.at[:, pl.ds(s0, CS)], outsem.at[k]
        )

    def start_in(ci, k):
        for c in in_copies(ci, k):
            c.start()

    def wait_in(ci, k):
        for c in in_copies(ci, k):
            c.wait()

    def compute(k):
        @plsc.parallel_loop(0, VPR, step=1, unroll=4)
        def vec(j):
            off = j * LANES
            for s in range(CS):
                e = ebuf[k, s, pl.ds(off, LANES)]
                for b in range(BATCH):
                    xbuf[k, b, s, pl.ds(off, LANES)] = (
                        xbuf[k, b, s, pl.ds(off, LANES)] + e
                    )

    start_in(0, 0)

    def step(ci, carry):
        k = lax.rem(ci, NBUF)
        kn = lax.rem(ci + 1, NBUF)

        @pl.when(jnp.logical_and(ci + 1 < NCHUNK, ci >= NBUF - 1))
        def _():
            out_copy(ci + 1 - NBUF, kn).wait()

        @pl.when(ci + 1 < NCHUNK)
        def _():
            start_in(ci + 1, kn)

        wait_in(ci, k)
        compute(k)
        out_copy(ci, k).start()
        return carry

    lax.fori_loop(0, NCHUNK, step, 0)
    for ci in range(NCHUNK - NBUF, NCHUNK):
        out_copy(ci, ci % NBUF).wait()


INTERPRET = False


def kernel(x, embedding):
    mesh = plsc.VectorSubcoreMesh(
        core_axis_name="c", subcore_axis_name="s", num_cores=NC, num_subcores=NS
    )
    f = pl.kernel(
        _sc_body,
        jax.ShapeDtypeStruct((BATCH, SEQ, D), jnp.float32),
        mesh=mesh,
        scratch_types=[
            pltpu.VMEM((NBUF, BATCH, CS, D), jnp.float32),
            pltpu.VMEM((NBUF, CS, D), jnp.float32),
            pltpu.SemaphoreType.DMA((NBUF,)),
            pltpu.SemaphoreType.DMA((NBUF,)),
        ],
        interpret=INTERPRET,
    )
    return f(x, embedding)
